# baseline (speedup 1.0000x reference)
import functools

import jax
import jax.numpy as jnp
from jax import lax
from jax.experimental import pallas as pl
from jax.experimental.pallas import tpu as pltpu
from jax.experimental.pallas import tpu_sc as plsc

N = 10000
E = 320000
L = 16
V = 50000
D = 128
G = 64

NC = 2
NS = 16
NW = NC * NS

N_PAD = 10240
NPC = N_PAD // NC
NPT = N_PAD // NW
TCHUNK = 96
TOK_CH = 54
TOK_PT = TOK_CH * TCHUNK

ECHUNK = 88
EPT_CH = 115
EPT = EPT_CH * ECHUNK
E_PAD = NW * EPT

GPT = G // NW

N_ACC = 10112
RPT_E = N_ACC // NS
DUMP_ROW = N

_EMB_RING = 6
_EMB_AHEAD = 3

_EDGE_RBUF = 4
_EDGE_IBUF = 5


def _gs_pipeline(table_hbm, src_v, dst_v, acc_sh, bufs, gsem, ssem,
                 n_chunks, ring, ahead):
  gd = [None] * n_chunks
  sd = [None] * n_chunks
  s_waited = [False] * n_chunks

  def gather(j):
    gd[j] = pltpu.async_copy(
        table_hbm.at[src_v.at[j]], bufs.at[j % ring], gsem.at[j % ring])

  for j in range(min(ahead, n_chunks)):
    gather(j)
  for j in range(n_chunks):
    gd[j].wait()
    sd[j] = pltpu.async_copy(
        bufs.at[j % ring], acc_sh.at[dst_v.at[j]], ssem.at[j % ring],
        add=True)
    jg = j + ahead
    if jg < n_chunks:
      jp = jg - ring
      if jp >= 0 and not s_waited[jp]:
        sd[jp].wait()
        s_waited[jp] = True
      gather(jg)
  for k in range(n_chunks):
    if sd[k] is not None and not s_waited[k]:
      sd[k].wait()
      s_waited[k] = True


def _zero_rows(rows_v, n_rows):
  z = jnp.zeros((16,), jnp.float32)

  def body(i, _):
    for k in range(8):
      rows_v[i, pl.ds(k * 16, 16)] = z
    return 0

  lax.fori_loop(0, n_rows, body, 0)


def _emb_body(emb_hbm, tok_hbm, dst_hbm, out_hbm,
              tok_v, dst_v, bufs, acc_sh, gsem, ssem):
  c = lax.axis_index("c")
  s = lax.axis_index("s")

  pltpu.sync_copy(tok_hbm.at[c, s], tok_v)
  pltpu.sync_copy(dst_hbm.at[s], dst_v)

  _zero_rows(bufs.at[0], TCHUNK)
  for q in range(3):
    pltpu.sync_copy(bufs.at[0],
                    acc_sh.at[pl.ds(s * NPT + q * TCHUNK, TCHUNK)])
  pltpu.sync_copy(bufs.at[0, pl.ds(0, NPT - 3 * TCHUNK)],
                  acc_sh.at[pl.ds(s * NPT + 3 * TCHUNK, NPT - 3 * TCHUNK)])
  plsc.subcore_barrier()

  _gs_pipeline(emb_hbm, tok_v, dst_v, acc_sh, bufs, gsem, ssem,
               TOK_CH, _EMB_RING, _EMB_AHEAD)

  plsc.subcore_barrier()
  pltpu.sync_copy(acc_sh.at[pl.ds(s * NPT, NPT)],
                  out_hbm.at[pl.ds(c * NPC + s * NPT, NPT)])


def _edge_body(h_hbm, idx_hbm, out_hbm, idx_v, bufs, acc_sh,
               isem, gsem, ssem):
  c = lax.axis_index("c")
  s = lax.axis_index("s")

  _zero_rows(bufs.at[0], ECHUNK)
  for q in range(7):
    pltpu.sync_copy(bufs.at[0],
                    acc_sh.at[pl.ds(s * RPT_E + q * ECHUNK, ECHUNK)])
  pltpu.sync_copy(bufs.at[0, pl.ds(0, RPT_E - 7 * ECHUNK)],
                  acc_sh.at[pl.ds(s * RPT_E + 7 * ECHUNK,
                                  RPT_E - 7 * ECHUNK)])
  plsc.subcore_barrier()

  n = EPT_CH
  idd = [None] * n
  gd = [None] * n
  sd = [None] * n
  s_waited = [False] * n

  def issue_idx(k):
    idd[k] = pltpu.async_copy(
        idx_hbm.at[c, s, k], idx_v.at[k % _EDGE_IBUF],
        isem.at[k % _EDGE_IBUF])

  def issue_gather(k):
    idd[k].wait()
    gd[k] = pltpu.async_copy(
        h_hbm.at[idx_v.at[k % _EDGE_IBUF, 0]], bufs.at[k % _EDGE_RBUF],
        gsem.at[k % _EDGE_RBUF])

  def issue_scatter(k):
    sd[k] = pltpu.async_copy(
        bufs.at[k % _EDGE_RBUF], acc_sh.at[idx_v.at[k % _EDGE_IBUF, 1]],
        ssem.at[k % _EDGE_RBUF], add=True)

  for k in range(min(4, n)):
    issue_idx(k)
  for k in range(min(3, n)):
    issue_gather(k)
  for j in range(n):
    gd[j].wait()
    issue_scatter(j)
    jg = j + 3
    if jg < n:
      jp = jg - _EDGE_RBUF
      if jp >= 0:
        sd[jp].wait()
        s_waited[jp] = True
      issue_gather(jg)
    ji = j + 4
    if ji < n:
      issue_idx(ji)
  for k in range(n):
    if sd[k] is not None and not s_waited[k]:
      sd[k].wait()

  plsc.subcore_barrier()
  pltpu.sync_copy(acc_sh.at[pl.ds(s * RPT_E, RPT_E)],
                  out_hbm.at[c, pl.ds(s * RPT_E, RPT_E)])


def _mm_body(a0_ref, a1_ref, w_ref, b_ref, o_ref):
  acc = a0_ref[...] + a1_ref[...]
  y = jnp.dot(acc, w_ref[...], preferred_element_type=jnp.float32)
  o_ref[...] = jnp.maximum(y + b_ref[...], 0.0)


_MM_BLK = 512


def _mm_relu(a0, a1, w, b):
  return pl.pallas_call(
      _mm_body,
      grid=(N_PAD // _MM_BLK,),
      in_specs=[
          pl.BlockSpec((_MM_BLK, D), lambda i: (i, 0)),
          pl.BlockSpec((_MM_BLK, D), lambda i: (i, 0)),
          pl.BlockSpec((D, D), lambda i: (0, 0)),
          pl.BlockSpec((1, D), lambda i: (0, 0)),
      ],
      out_specs=pl.BlockSpec((_MM_BLK, D), lambda i: (i, 0)),
      out_shape=jax.ShapeDtypeStruct((N_PAD, D), jnp.float32),
  )(a0, a1, w, b.reshape(1, D))


def _pool_body(h_hbm, batch_hbm, out_hbm, batch_v, rows_v, out_v, sem):
  c = lax.axis_index("c")
  s = lax.axis_index("s")
  wid = c * NS + s
  g0 = wid * GPT

  pltpu.sync_copy(batch_hbm, batch_v)

  iota = lax.iota(jnp.int32, 16)

  def lower_bound(g):
    def body(_, lohi):
      lo, hi = lohi
      mid = (lo + hi) // 2
      less = batch_v[pl.ds(mid, 16)][0] < g
      return (jnp.where(less, mid + 1, lo), jnp.where(less, hi, mid))

    lo, _ = lax.fori_loop(0, 14, body,
                          (jnp.int32(0), jnp.int32(N_PAD)))
    return lo

  bounds = [lower_bound(g0 + t) for t in range(GPT + 1)]

  neg_inf = jnp.full((16,), -jnp.inf, jnp.float32)

  for t in range(GPT):
    start = bounds[t]
    end = bounds[t + 1]
    nch = (end - start + 15) // 16

    def chunk_body(ci, acc8, start=start, end=end):
      idx = jnp.minimum(start + ci * 16 + iota, end - 1)
      pltpu.async_copy(h_hbm.at[idx], rows_v, sem).wait()
      out = list(acc8)
      for k in range(8):
        m = out[k]
        for r in range(16):
          m = jnp.maximum(m, rows_v[r, pl.ds(k * 16, 16)])
        out[k] = m
      return tuple(out)

    acc8 = lax.fori_loop(0, nch, chunk_body,
                         tuple(neg_inf for _ in range(8)))
    for k in range(8):
      out_v[t, pl.ds(k * 16, 16)] = acc8[k]

  pltpu.sync_copy(out_v, out_hbm.at[pl.ds(g0, GPT)])


@functools.lru_cache(maxsize=None)
def _sc_kernels():
  mesh = plsc.VectorSubcoreMesh(
      core_axis_name="c", subcore_axis_name="s",
      num_cores=NC, num_subcores=NS)
  emb = pl.kernel(
      _emb_body,
      out_type=jax.ShapeDtypeStruct((N_PAD, D), jnp.float32),
      mesh=mesh,
      scratch_types=[
          pltpu.VMEM((TOK_CH, TCHUNK), jnp.int32),
          pltpu.VMEM((TOK_CH, TCHUNK), jnp.int32),
          pltpu.VMEM((_EMB_RING, TCHUNK, D), jnp.float32),
          pltpu.VMEM_SHARED((NPC, D), jnp.float32),
          pltpu.SemaphoreType.DMA((_EMB_RING,)),
          pltpu.SemaphoreType.DMA((_EMB_RING,)),
      ],
  )
  edge = pl.kernel(
      _edge_body,
      out_type=jax.ShapeDtypeStruct((NC, N_PAD, D), jnp.float32),
      mesh=mesh,
      scratch_types=[
          pltpu.VMEM((_EDGE_IBUF, 2, ECHUNK), jnp.int32),
          pltpu.VMEM((_EDGE_RBUF, ECHUNK, D), jnp.float32),
          pltpu.VMEM_SHARED((N_ACC, D), jnp.float32),
          pltpu.SemaphoreType.DMA((_EDGE_IBUF,)),
          pltpu.SemaphoreType.DMA((_EDGE_RBUF,)),
          pltpu.SemaphoreType.DMA((_EDGE_RBUF,)),
      ],
  )
  pool = pl.kernel(
      _pool_body,
      out_type=jax.ShapeDtypeStruct((G, D), jnp.float32),
      mesh=mesh,
      scratch_types=[
          pltpu.VMEM((N_PAD + 16,), jnp.int32),
          pltpu.VMEM((16, D), jnp.float32),
          pltpu.VMEM((GPT, D), jnp.float32),
          pltpu.SemaphoreType.DMA,
      ],
  )
  return emb, edge, pool


def kernel(x, edge_index, batch, emb_table, W0, b0, W1, b1):
  x = x.astype(jnp.int32)
  edge_index = edge_index.astype(jnp.int32)
  batch = batch.astype(jnp.int32)

  x_pad = jnp.zeros((N_PAD, L), jnp.int32).at[:N].set(x)
  tpad = TOK_PT - NPT * L
  tok = jnp.concatenate(
      [x_pad.reshape(NC, NS, NPT, L).transpose(0, 1, 3, 2).reshape(
          NC, NS, NPT * L),
       jnp.zeros((NC, NS, tpad), jnp.int32)], axis=2).reshape(
           NC, NS, TOK_CH, TCHUNK)
  dpat = jnp.concatenate([jnp.tile(jnp.arange(NPT, dtype=jnp.int32), L),
                          jnp.zeros((tpad,), jnp.int32)])
  emb_dst = (dpat[None, :]
             + (jnp.arange(NS, dtype=jnp.int32) * NPT)[:, None]).reshape(
                 NS, TOK_CH, TCHUNK)

  src = jnp.zeros((E_PAD,), jnp.int32).at[:E].set(edge_index[0])
  dst = jnp.full((E_PAD,), DUMP_ROW, jnp.int32).at[:E].set(edge_index[1])
  eidx = jnp.stack([src.reshape(NC, NS, EPT_CH, ECHUNK),
                    dst.reshape(NC, NS, EPT_CH, ECHUNK)], axis=3)

  batch_pad = jnp.full((N_PAD + 16,), G, jnp.int32).at[:N].set(batch)

  emb_k, edge_k, pool_k = _sc_kernels()
  h0 = emb_k(emb_table, tok, emb_dst)
  agg = edge_k(h0, eidx)
  h1 = _mm_relu(agg[0], agg[1], W0, b0)
  agg = edge_k(h1, eidx)
  h2 = _mm_relu(agg[0], agg[1], W1, b1)
  pooled = pool_k(h2, batch_pad)
  return pooled


# speedup vs baseline: 1.0981x; 1.0981x over previous
import functools

import jax
import jax.numpy as jnp
from jax import lax
from jax.experimental import pallas as pl
from jax.experimental.pallas import tpu as pltpu
from jax.experimental.pallas import tpu_sc as plsc

N = 10000
E = 320000
L = 16
V = 50000
D = 128
G = 64

NC = 2
NS = 16
NW = NC * NS

N_PAD = 10240
NPC = N_PAD // NC
NPT = N_PAD // NW
TCHUNK = 96
TOK_CH = 54
TOK_PT = TOK_CH * TCHUNK

ECHUNK = 120
EPT_CH = 84
EPT = EPT_CH * ECHUNK
E_PAD = NW * EPT

GPT = G // NW

N_ACC = 10112
RPT_E = N_ACC // NS
DUMP_ROW = N

_EMB_RING = 6
_EMB_AHEAD = 3

_EDGE_RBUF = 3
_EDGE_IBUF = 4


def _gs_pipeline(table_hbm, src_v, dst_v, acc_sh, bufs, gsem, ssem,
                 n_chunks, ring, ahead):
  gd = [None] * n_chunks
  sd = [None] * n_chunks
  s_waited = [False] * n_chunks

  def gather(j):
    gd[j] = pltpu.async_copy(
        table_hbm.at[src_v.at[j]], bufs.at[j % ring], gsem.at[j % ring])

  for j in range(min(ahead, n_chunks)):
    gather(j)
  for j in range(n_chunks):
    gd[j].wait()
    sd[j] = pltpu.async_copy(
        bufs.at[j % ring], acc_sh.at[dst_v.at[j]], ssem.at[j % ring],
        add=True)
    jg = j + ahead
    if jg < n_chunks:
      jp = jg - ring
      if jp >= 0 and not s_waited[jp]:
        sd[jp].wait()
        s_waited[jp] = True
      gather(jg)
  for k in range(n_chunks):
    if sd[k] is not None and not s_waited[k]:
      sd[k].wait()
      s_waited[k] = True


def _zero_rows(rows_v, n_rows):
  z = jnp.zeros((16,), jnp.float32)

  def body(i, _):
    for k in range(8):
      rows_v[i, pl.ds(k * 16, 16)] = z
    return 0

  lax.fori_loop(0, n_rows, body, 0)


def _emb_body(emb_hbm, tok_hbm, dst_hbm, out_hbm,
              tok_v, dst_v, bufs, acc_sh, gsem, ssem):
  c = lax.axis_index("c")
  s = lax.axis_index("s")

  pltpu.sync_copy(tok_hbm.at[c, s], tok_v)
  pltpu.sync_copy(dst_hbm.at[s], dst_v)

  _zero_rows(bufs.at[0], TCHUNK)
  for q in range(3):
    pltpu.sync_copy(bufs.at[0],
                    acc_sh.at[pl.ds(s * NPT + q * TCHUNK, TCHUNK)])
  pltpu.sync_copy(bufs.at[0, pl.ds(0, NPT - 3 * TCHUNK)],
                  acc_sh.at[pl.ds(s * NPT + 3 * TCHUNK, NPT - 3 * TCHUNK)])
  plsc.subcore_barrier()

  _gs_pipeline(emb_hbm, tok_v, dst_v, acc_sh, bufs, gsem, ssem,
               TOK_CH, _EMB_RING, _EMB_AHEAD)

  plsc.subcore_barrier()
  pltpu.sync_copy(acc_sh.at[pl.ds(s * NPT, NPT)],
                  out_hbm.at[pl.ds(c * NPC + s * NPT, NPT)])


def _edge_body(h_hbm, idx_hbm, out_hbm, idx_v, bufs, acc_sh,
               isem, gsem, ssem):
  c = lax.axis_index("c")
  s = lax.axis_index("s")

  _zero_rows(bufs.at[0], ECHUNK)
  for q in range(5):
    pltpu.sync_copy(bufs.at[0],
                    acc_sh.at[pl.ds(s * RPT_E + q * ECHUNK, ECHUNK)])
  pltpu.sync_copy(bufs.at[0, pl.ds(0, RPT_E - 5 * ECHUNK)],
                  acc_sh.at[pl.ds(s * RPT_E + 5 * ECHUNK,
                                  RPT_E - 5 * ECHUNK)])
  plsc.subcore_barrier()

  n = EPT_CH
  idd = [None] * n
  gd = [None] * n
  sd = [None] * n
  s_waited = [False] * n

  def issue_idx(k):
    idd[k] = pltpu.async_copy(
        idx_hbm.at[c, s, k], idx_v.at[k % _EDGE_IBUF],
        isem.at[k % _EDGE_IBUF])

  def issue_gather(k):
    idd[k].wait()
    gd[k] = pltpu.async_copy(
        h_hbm.at[idx_v.at[k % _EDGE_IBUF, 0]], bufs.at[k % _EDGE_RBUF],
        gsem.at[k % _EDGE_RBUF])

  def issue_scatter(k):
    sd[k] = pltpu.async_copy(
        bufs.at[k % _EDGE_RBUF], acc_sh.at[idx_v.at[k % _EDGE_IBUF, 1]],
        ssem.at[k % _EDGE_RBUF], add=True)

  for k in range(min(3, n)):
    issue_idx(k)
  for k in range(min(2, n)):
    issue_gather(k)
  for j in range(n):
    gd[j].wait()
    issue_scatter(j)
    jg = j + 2
    if jg < n:
      jp = jg - _EDGE_RBUF
      if jp >= 0:
        sd[jp].wait()
        s_waited[jp] = True
      issue_gather(jg)
    ji = j + 3
    if ji < n:
      issue_idx(ji)
  for k in range(n):
    if sd[k] is not None and not s_waited[k]:
      sd[k].wait()

  plsc.subcore_barrier()
  pltpu.sync_copy(acc_sh.at[pl.ds(s * RPT_E, RPT_E)],
                  out_hbm.at[c, pl.ds(s * RPT_E, RPT_E)])


def _mm_body(a0_ref, a1_ref, w_ref, b_ref, o_ref):
  acc = a0_ref[...] + a1_ref[...]
  y = jnp.dot(acc, w_ref[...], preferred_element_type=jnp.float32)
  o_ref[...] = jnp.maximum(y + b_ref[...], 0.0)


_MM_BLK = 512


def _mm_relu(a0, a1, w, b):
  return pl.pallas_call(
      _mm_body,
      grid=(N_PAD // _MM_BLK,),
      in_specs=[
          pl.BlockSpec((_MM_BLK, D), lambda i: (i, 0)),
          pl.BlockSpec((_MM_BLK, D), lambda i: (i, 0)),
          pl.BlockSpec((D, D), lambda i: (0, 0)),
          pl.BlockSpec((1, D), lambda i: (0, 0)),
      ],
      out_specs=pl.BlockSpec((_MM_BLK, D), lambda i: (i, 0)),
      out_shape=jax.ShapeDtypeStruct((N_PAD, D), jnp.float32),
  )(a0, a1, w, b.reshape(1, D))


def _pool_body(h_hbm, batch_hbm, out_hbm, batch_v, rows_v, out_v, sem):
  c = lax.axis_index("c")
  s = lax.axis_index("s")
  wid = c * NS + s
  g0 = wid * GPT

  pltpu.sync_copy(batch_hbm, batch_v)

  iota = lax.iota(jnp.int32, 16)

  def lower_bound(g):
    def body(_, lohi):
      lo, hi = lohi
      mid = (lo + hi) // 2
      less = batch_v[pl.ds(mid, 16)][0] < g
      return (jnp.where(less, mid + 1, lo), jnp.where(less, hi, mid))

    lo, _ = lax.fori_loop(0, 14, body,
                          (jnp.int32(0), jnp.int32(N_PAD)))
    return lo

  bounds = [lower_bound(g0 + t) for t in range(GPT + 1)]

  neg_inf = jnp.full((16,), -jnp.inf, jnp.float32)

  for t in range(GPT):
    start = bounds[t]
    end = bounds[t + 1]
    nch = (end - start + 15) // 16

    def chunk_body(ci, acc8, start=start, end=end):
      idx = jnp.minimum(start + ci * 16 + iota, end - 1)
      pltpu.async_copy(h_hbm.at[idx], rows_v, sem).wait()
      out = list(acc8)
      for k in range(8):
        m = out[k]
        for r in range(16):
          m = jnp.maximum(m, rows_v[r, pl.ds(k * 16, 16)])
        out[k] = m
      return tuple(out)

    acc8 = lax.fori_loop(0, nch, chunk_body,
                         tuple(neg_inf for _ in range(8)))
    for k in range(8):
      out_v[t, pl.ds(k * 16, 16)] = acc8[k]

  pltpu.sync_copy(out_v, out_hbm.at[pl.ds(g0, GPT)])


@functools.lru_cache(maxsize=None)
def _sc_kernels():
  mesh = plsc.VectorSubcoreMesh(
      core_axis_name="c", subcore_axis_name="s",
      num_cores=NC, num_subcores=NS)
  emb = pl.kernel(
      _emb_body,
      out_type=jax.ShapeDtypeStruct((N_PAD, D), jnp.float32),
      mesh=mesh,
      scratch_types=[
          pltpu.VMEM((TOK_CH, TCHUNK), jnp.int32),
          pltpu.VMEM((TOK_CH, TCHUNK), jnp.int32),
          pltpu.VMEM((_EMB_RING, TCHUNK, D), jnp.float32),
          pltpu.VMEM_SHARED((NPC, D), jnp.float32),
          pltpu.SemaphoreType.DMA((_EMB_RING,)),
          pltpu.SemaphoreType.DMA((_EMB_RING,)),
      ],
  )
  edge = pl.kernel(
      _edge_body,
      out_type=jax.ShapeDtypeStruct((NC, N_PAD, D), jnp.float32),
      mesh=mesh,
      scratch_types=[
          pltpu.VMEM((_EDGE_IBUF, 2, ECHUNK), jnp.int32),
          pltpu.VMEM((_EDGE_RBUF, ECHUNK, D), jnp.float32),
          pltpu.VMEM_SHARED((N_ACC, D), jnp.float32),
          pltpu.SemaphoreType.DMA((_EDGE_IBUF,)),
          pltpu.SemaphoreType.DMA((_EDGE_RBUF,)),
          pltpu.SemaphoreType.DMA((_EDGE_RBUF,)),
      ],
  )
  pool = pl.kernel(
      _pool_body,
      out_type=jax.ShapeDtypeStruct((G, D), jnp.float32),
      mesh=mesh,
      scratch_types=[
          pltpu.VMEM((N_PAD + 16,), jnp.int32),
          pltpu.VMEM((16, D), jnp.float32),
          pltpu.VMEM((GPT, D), jnp.float32),
          pltpu.SemaphoreType.DMA,
      ],
  )
  return emb, edge, pool


def kernel(x, edge_index, batch, emb_table, W0, b0, W1, b1):
  x = x.astype(jnp.int32)
  edge_index = edge_index.astype(jnp.int32)
  batch = batch.astype(jnp.int32)

  x_pad = jnp.zeros((N_PAD, L), jnp.int32).at[:N].set(x)
  tpad = TOK_PT - NPT * L
  tok = jnp.concatenate(
      [x_pad.reshape(NC, NS, NPT, L).transpose(0, 1, 3, 2).reshape(
          NC, NS, NPT * L),
       jnp.zeros((NC, NS, tpad), jnp.int32)], axis=2).reshape(
           NC, NS, TOK_CH, TCHUNK)
  dpat = jnp.concatenate([jnp.tile(jnp.arange(NPT, dtype=jnp.int32), L),
                          jnp.zeros((tpad,), jnp.int32)])
  emb_dst = (dpat[None, :]
             + (jnp.arange(NS, dtype=jnp.int32) * NPT)[:, None]).reshape(
                 NS, TOK_CH, TCHUNK)

  src = jnp.zeros((E_PAD,), jnp.int32).at[:E].set(edge_index[0])
  dst = jnp.full((E_PAD,), DUMP_ROW, jnp.int32).at[:E].set(edge_index[1])
  eidx = jnp.stack([src.reshape(NC, NS, EPT_CH, ECHUNK),
                    dst.reshape(NC, NS, EPT_CH, ECHUNK)], axis=3)

  batch_pad = jnp.full((N_PAD + 16,), G, jnp.int32).at[:N].set(batch)

  emb_k, edge_k, pool_k = _sc_kernels()
  h0 = emb_k(emb_table, tok, emb_dst)
  agg = edge_k(h0, eidx)
  h1 = _mm_relu(agg[0], agg[1], W0, b0)
  agg = edge_k(h1, eidx)
  h2 = _mm_relu(agg[0], agg[1], W1, b1)
  pooled = pool_k(h2, batch_pad)
  return pooled


# speedup vs baseline: 1.2278x; 1.1181x over previous
import functools

import jax
import jax.numpy as jnp
from jax import lax
from jax.experimental import pallas as pl
from jax.experimental.pallas import tpu as pltpu
from jax.experimental.pallas import tpu_sc as plsc

N = 10000
E = 320000
L = 16
V = 50000
D = 128
G = 64

NC = 2
NS = 16
NW = NC * NS

N_PAD = 10240
NPC = N_PAD // NC
NPT = N_PAD // NW
TCHUNK = 128
TOK_CH = 40
TOK_PT = TOK_CH * TCHUNK

ECHUNK = 120
EPT_CH = 84
EPT = EPT_CH * ECHUNK
E_PAD = NW * EPT

GPT = G // NW

N_ACC = 10112
RPT_E = N_ACC // NS
DUMP_ROW = N

_EMB_RING = 4
_EMB_AHEAD = 2

_EDGE_RBUF = 3
_EDGE_IBUF = 4


def _gs_pipeline(table_hbm, src_v, dst_v, acc_sh, bufs, gsem, ssem,
                 n_chunks, ring, ahead):
  gd = [None] * n_chunks
  sd = [None] * n_chunks
  s_waited = [False] * n_chunks

  def gather(j):
    gd[j] = pltpu.async_copy(
        table_hbm.at[src_v.at[j]], bufs.at[j % ring], gsem.at[j % ring])

  for j in range(min(ahead, n_chunks)):
    gather(j)
  for j in range(n_chunks):
    gd[j].wait()
    sd[j] = pltpu.async_copy(
        bufs.at[j % ring], acc_sh.at[dst_v.at[j]], ssem.at[j % ring],
        add=True)
    jg = j + ahead
    if jg < n_chunks:
      jp = jg - ring
      if jp >= 0 and not s_waited[jp]:
        sd[jp].wait()
        s_waited[jp] = True
      gather(jg)
  for k in range(n_chunks):
    if sd[k] is not None and not s_waited[k]:
      sd[k].wait()
      s_waited[k] = True


def _zero_rows(rows_v, n_rows):
  z = jnp.zeros((16,), jnp.float32)

  def body(i, _):
    for k in range(8):
      rows_v[i, pl.ds(k * 16, 16)] = z
    return 0

  lax.fori_loop(0, n_rows, body, 0)


def _emb_body(emb_hbm, tok_hbm, dst_hbm, out_hbm,
              tok_v, dst_v, bufs, acc_sh, gsem, ssem):
  c = lax.axis_index("c")
  s = lax.axis_index("s")

  pltpu.sync_copy(tok_hbm.at[c, s], tok_v)
  pltpu.sync_copy(dst_hbm.at[s], dst_v)

  _zero_rows(bufs.at[0], TCHUNK)
  for q in range(2):
    pltpu.sync_copy(bufs.at[0],
                    acc_sh.at[pl.ds(s * NPT + q * TCHUNK, TCHUNK)])
  pltpu.sync_copy(bufs.at[0, pl.ds(0, NPT - 2 * TCHUNK)],
                  acc_sh.at[pl.ds(s * NPT + 2 * TCHUNK, NPT - 2 * TCHUNK)])
  plsc.subcore_barrier()

  _gs_pipeline(emb_hbm, tok_v, dst_v, acc_sh, bufs, gsem, ssem,
               TOK_CH, _EMB_RING, _EMB_AHEAD)

  plsc.subcore_barrier()
  pltpu.sync_copy(acc_sh.at[pl.ds(s * NPT, NPT)],
                  out_hbm.at[pl.ds(c * NPC + s * NPT, NPT)])


def _edge_body(h_hbm, idx_hbm, out_hbm, idx_v, bufs, acc_sh,
               isem, gsem, ssem):
  c = lax.axis_index("c")
  s = lax.axis_index("s")

  _zero_rows(bufs.at[0], ECHUNK)
  for q in range(5):
    pltpu.sync_copy(bufs.at[0],
                    acc_sh.at[pl.ds(s * RPT_E + q * ECHUNK, ECHUNK)])
  pltpu.sync_copy(bufs.at[0, pl.ds(0, RPT_E - 5 * ECHUNK)],
                  acc_sh.at[pl.ds(s * RPT_E + 5 * ECHUNK,
                                  RPT_E - 5 * ECHUNK)])
  plsc.subcore_barrier()

  n = EPT_CH
  idd = [None] * n
  gd = [None] * n
  sd = [None] * n
  s_waited = [False] * n

  def issue_idx(k):
    idd[k] = pltpu.async_copy(
        idx_hbm.at[c, s, k], idx_v.at[k % _EDGE_IBUF],
        isem.at[k % _EDGE_IBUF])

  def issue_gather(k):
    idd[k].wait()
    gd[k] = pltpu.async_copy(
        h_hbm.at[idx_v.at[k % _EDGE_IBUF, 0]], bufs.at[k % _EDGE_RBUF],
        gsem.at[k % _EDGE_RBUF])

  def issue_scatter(k):
    sd[k] = pltpu.async_copy(
        bufs.at[k % _EDGE_RBUF], acc_sh.at[idx_v.at[k % _EDGE_IBUF, 1]],
        ssem.at[k % _EDGE_RBUF], add=True)

  for k in range(min(3, n)):
    issue_idx(k)
  for k in range(min(2, n)):
    issue_gather(k)
  for j in range(n):
    gd[j].wait()
    issue_scatter(j)
    jg = j + 2
    if jg < n:
      jp = jg - _EDGE_RBUF
      if jp >= 0:
        sd[jp].wait()
        s_waited[jp] = True
      issue_gather(jg)
    ji = j + 3
    if ji < n:
      issue_idx(ji)
  for k in range(n):
    if sd[k] is not None and not s_waited[k]:
      sd[k].wait()

  plsc.subcore_barrier()
  pltpu.sync_copy(acc_sh.at[pl.ds(s * RPT_E, RPT_E)],
                  out_hbm.at[c, pl.ds(s * RPT_E, RPT_E)])


def _mm_body(a0_ref, a1_ref, w_ref, b_ref, o_ref):
  acc = a0_ref[...] + a1_ref[...]
  y = jnp.dot(acc, w_ref[...], preferred_element_type=jnp.float32)
  o_ref[...] = jnp.maximum(y + b_ref[...], 0.0)


_MM_BLK = 512


def _mm_relu(a0, a1, w, b):
  return pl.pallas_call(
      _mm_body,
      grid=(N_PAD // _MM_BLK,),
      in_specs=[
          pl.BlockSpec((_MM_BLK, D), lambda i: (i, 0)),
          pl.BlockSpec((_MM_BLK, D), lambda i: (i, 0)),
          pl.BlockSpec((D, D), lambda i: (0, 0)),
          pl.BlockSpec((1, D), lambda i: (0, 0)),
      ],
      out_specs=pl.BlockSpec((_MM_BLK, D), lambda i: (i, 0)),
      out_shape=jax.ShapeDtypeStruct((N_PAD, D), jnp.float32),
  )(a0, a1, w, b.reshape(1, D))


def _pool_body(h_hbm, batch_hbm, out_hbm, batch_v, rows_v, out_v, sem):
  c = lax.axis_index("c")
  s = lax.axis_index("s")
  wid = c * NS + s
  g0 = wid * GPT

  pltpu.sync_copy(batch_hbm, batch_v)

  iota = lax.iota(jnp.int32, 16)

  def lower_bound(g):
    def body(_, lohi):
      lo, hi = lohi
      mid = (lo + hi) // 2
      less = batch_v[pl.ds(mid, 16)][0] < g
      return (jnp.where(less, mid + 1, lo), jnp.where(less, hi, mid))

    lo, _ = lax.fori_loop(0, 14, body,
                          (jnp.int32(0), jnp.int32(N_PAD)))
    return lo

  bounds = [lower_bound(g0 + t) for t in range(GPT + 1)]

  neg_inf = jnp.full((16,), -jnp.inf, jnp.float32)

  for t in range(GPT):
    start = bounds[t]
    end = bounds[t + 1]
    nch = (end - start + 15) // 16

    def chunk_body(ci, acc8, start=start, end=end):
      idx = jnp.minimum(start + ci * 16 + iota, end - 1)
      pltpu.async_copy(h_hbm.at[idx], rows_v, sem).wait()
      out = list(acc8)
      for k in range(8):
        m = out[k]
        for r in range(16):
          m = jnp.maximum(m, rows_v[r, pl.ds(k * 16, 16)])
        out[k] = m
      return tuple(out)

    acc8 = lax.fori_loop(0, nch, chunk_body,
                         tuple(neg_inf for _ in range(8)))
    for k in range(8):
      out_v[t, pl.ds(k * 16, 16)] = acc8[k]

  pltpu.sync_copy(out_v, out_hbm.at[pl.ds(g0, GPT)])


@functools.lru_cache(maxsize=None)
def _sc_kernels():
  mesh = plsc.VectorSubcoreMesh(
      core_axis_name="c", subcore_axis_name="s",
      num_cores=NC, num_subcores=NS)
  emb = pl.kernel(
      _emb_body,
      out_type=jax.ShapeDtypeStruct((N_PAD, D), jnp.float32),
      mesh=mesh,
      scratch_types=[
          pltpu.VMEM((TOK_CH, TCHUNK), jnp.int32),
          pltpu.VMEM((TOK_CH, TCHUNK), jnp.int32),
          pltpu.VMEM((_EMB_RING, TCHUNK, D), jnp.float32),
          pltpu.VMEM_SHARED((NPC, D), jnp.float32),
          pltpu.SemaphoreType.DMA((_EMB_RING,)),
          pltpu.SemaphoreType.DMA((_EMB_RING,)),
      ],
  )
  edge = pl.kernel(
      _edge_body,
      out_type=jax.ShapeDtypeStruct((NC, N_PAD, D), jnp.float32),
      mesh=mesh,
      scratch_types=[
          pltpu.VMEM((_EDGE_IBUF, 2, ECHUNK), jnp.int32),
          pltpu.VMEM((_EDGE_RBUF, ECHUNK, D), jnp.float32),
          pltpu.VMEM_SHARED((N_ACC, D), jnp.float32),
          pltpu.SemaphoreType.DMA((_EDGE_IBUF,)),
          pltpu.SemaphoreType.DMA((_EDGE_RBUF,)),
          pltpu.SemaphoreType.DMA((_EDGE_RBUF,)),
      ],
  )
  pool = pl.kernel(
      _pool_body,
      out_type=jax.ShapeDtypeStruct((G, D), jnp.float32),
      mesh=mesh,
      scratch_types=[
          pltpu.VMEM((N_PAD + 16,), jnp.int32),
          pltpu.VMEM((16, D), jnp.float32),
          pltpu.VMEM((GPT, D), jnp.float32),
          pltpu.SemaphoreType.DMA,
      ],
  )
  return emb, edge, pool


def kernel(x, edge_index, batch, emb_table, W0, b0, W1, b1):
  x = x.astype(jnp.int32)
  edge_index = edge_index.astype(jnp.int32)
  batch = batch.astype(jnp.int32)

  x_pad = jnp.zeros((N_PAD, L), jnp.int32).at[:N].set(x)
  tpad = TOK_PT - NPT * L
  tok = jnp.concatenate(
      [x_pad.reshape(NC, NS, NPT, L).transpose(0, 1, 3, 2).reshape(
          NC, NS, NPT * L),
       jnp.zeros((NC, NS, tpad), jnp.int32)], axis=2).reshape(
           NC, NS, TOK_CH, TCHUNK)
  dpat = jnp.concatenate([jnp.tile(jnp.arange(NPT, dtype=jnp.int32), L),
                          jnp.zeros((tpad,), jnp.int32)])
  emb_dst = (dpat[None, :]
             + (jnp.arange(NS, dtype=jnp.int32) * NPT)[:, None]).reshape(
                 NS, TOK_CH, TCHUNK)

  src = jnp.zeros((E_PAD,), jnp.int32).at[:E].set(edge_index[0])
  dst = jnp.full((E_PAD,), DUMP_ROW, jnp.int32).at[:E].set(edge_index[1])
  eidx = jnp.stack([src.reshape(NC, NS, EPT_CH, ECHUNK),
                    dst.reshape(NC, NS, EPT_CH, ECHUNK)], axis=3)

  batch_pad = jnp.full((N_PAD + 16,), G, jnp.int32).at[:N].set(batch)

  emb_k, edge_k, pool_k = _sc_kernels()
  h0 = emb_k(emb_table, tok, emb_dst)
  agg = edge_k(h0, eidx)
  h1 = _mm_relu(agg[0], agg[1], W0, b0)
  agg = edge_k(h1, eidx)
  h2 = _mm_relu(agg[0], agg[1], W1, b1)
  pooled = pool_k(h2, batch_pad)
  return pooled


# speedup vs baseline: 1.2289x; 1.0009x over previous
import functools

import jax
import jax.numpy as jnp
from jax import lax
from jax.experimental import pallas as pl
from jax.experimental.pallas import tpu as pltpu
from jax.experimental.pallas import tpu_sc as plsc

N = 10000
E = 320000
L = 16
V = 50000
D = 128
G = 64

NC = 2
NS = 16
NW = NC * NS

N_PAD = 10240
NPC = N_PAD // NC
NPT = N_PAD // NW
TCHUNK = 128
TOK_CH = 40
TOK_PT = TOK_CH * TCHUNK

ECHUNK = 120
EPT_CH = 84
EPT = EPT_CH * ECHUNK
E_PAD = NW * EPT

GPT = G // NW

N_ACC = 10112
RPT_E = N_ACC // NS
DUMP_ROW = N

_EMB_RING = 4
_EMB_AHEAD = 3

_EDGE_RBUF = 3
_EDGE_IBUF = 4


def _gs_pipeline(table_hbm, src_v, dst_v, acc_sh, bufs, gsem, ssem,
                 n_chunks, ring, ahead):
  gd = [None] * n_chunks
  sd = [None] * n_chunks
  s_waited = [False] * n_chunks

  def gather(j):
    gd[j] = pltpu.async_copy(
        table_hbm.at[src_v.at[j]], bufs.at[j % ring], gsem.at[j % ring])

  for j in range(min(ahead, n_chunks)):
    gather(j)
  for j in range(n_chunks):
    gd[j].wait()
    sd[j] = pltpu.async_copy(
        bufs.at[j % ring], acc_sh.at[dst_v.at[j]], ssem.at[j % ring],
        add=True)
    jg = j + ahead
    if jg < n_chunks:
      jp = jg - ring
      if jp >= 0 and not s_waited[jp]:
        sd[jp].wait()
        s_waited[jp] = True
      gather(jg)
  for k in range(n_chunks):
    if sd[k] is not None and not s_waited[k]:
      sd[k].wait()
      s_waited[k] = True


def _zero_rows(rows_v, n_rows):
  z = jnp.zeros((16,), jnp.float32)

  def body(i, _):
    for k in range(8):
      rows_v[i, pl.ds(k * 16, 16)] = z
    return 0

  lax.fori_loop(0, n_rows, body, 0)


def _emb_body(emb_hbm, tok_hbm, dst_hbm, out_hbm,
              tok_v, dst_v, bufs, acc_sh, gsem, ssem):
  c = lax.axis_index("c")
  s = lax.axis_index("s")

  pltpu.sync_copy(tok_hbm.at[c, s], tok_v)
  pltpu.sync_copy(dst_hbm.at[s], dst_v)

  _zero_rows(bufs.at[0], TCHUNK)
  for q in range(2):
    pltpu.sync_copy(bufs.at[0],
                    acc_sh.at[pl.ds(s * NPT + q * TCHUNK, TCHUNK)])
  pltpu.sync_copy(bufs.at[0, pl.ds(0, NPT - 2 * TCHUNK)],
                  acc_sh.at[pl.ds(s * NPT + 2 * TCHUNK, NPT - 2 * TCHUNK)])
  plsc.subcore_barrier()

  _gs_pipeline(emb_hbm, tok_v, dst_v, acc_sh, bufs, gsem, ssem,
               TOK_CH, _EMB_RING, _EMB_AHEAD)

  plsc.subcore_barrier()
  pltpu.sync_copy(acc_sh.at[pl.ds(s * NPT, NPT)],
                  out_hbm.at[pl.ds(c * NPC + s * NPT, NPT)])


def _edge_body(h_hbm, idx_hbm, out_hbm, idx_v, bufs, acc_sh,
               isem, gsem, ssem):
  c = lax.axis_index("c")
  s = lax.axis_index("s")

  _zero_rows(bufs.at[0], ECHUNK)
  for q in range(5):
    pltpu.sync_copy(bufs.at[0],
                    acc_sh.at[pl.ds(s * RPT_E + q * ECHUNK, ECHUNK)])
  pltpu.sync_copy(bufs.at[0, pl.ds(0, RPT_E - 5 * ECHUNK)],
                  acc_sh.at[pl.ds(s * RPT_E + 5 * ECHUNK,
                                  RPT_E - 5 * ECHUNK)])
  plsc.subcore_barrier()

  n = EPT_CH
  idd = [None] * n
  gd = [None] * n
  sd = [None] * n
  s_waited = [False] * n

  def issue_idx(k):
    idd[k] = pltpu.async_copy(
        idx_hbm.at[c, s, k], idx_v.at[k % _EDGE_IBUF],
        isem.at[k % _EDGE_IBUF])

  def issue_gather(k):
    idd[k].wait()
    gd[k] = pltpu.async_copy(
        h_hbm.at[idx_v.at[k % _EDGE_IBUF, 0]], bufs.at[k % _EDGE_RBUF],
        gsem.at[k % _EDGE_RBUF])

  def issue_scatter(k):
    sd[k] = pltpu.async_copy(
        bufs.at[k % _EDGE_RBUF], acc_sh.at[idx_v.at[k % _EDGE_IBUF, 1]],
        ssem.at[k % _EDGE_RBUF], add=True)

  for k in range(min(3, n)):
    issue_idx(k)
  for k in range(min(2, n)):
    issue_gather(k)
  for j in range(n):
    gd[j].wait()
    issue_scatter(j)
    jg = j + 2
    if jg < n:
      jp = jg - _EDGE_RBUF
      if jp >= 0:
        sd[jp].wait()
        s_waited[jp] = True
      issue_gather(jg)
    ji = j + 3
    if ji < n:
      issue_idx(ji)
  for k in range(n):
    if sd[k] is not None and not s_waited[k]:
      sd[k].wait()

  plsc.subcore_barrier()
  pltpu.sync_copy(acc_sh.at[pl.ds(s * RPT_E, RPT_E)],
                  out_hbm.at[c, pl.ds(s * RPT_E, RPT_E)])


def _mm_body(a0_ref, a1_ref, w_ref, b_ref, o_ref):
  acc = a0_ref[...] + a1_ref[...]
  y = jnp.dot(acc, w_ref[...], preferred_element_type=jnp.float32)
  o_ref[...] = jnp.maximum(y + b_ref[...], 0.0)


_MM_BLK = 512


def _mm_relu(a0, a1, w, b):
  return pl.pallas_call(
      _mm_body,
      grid=(N_PAD // _MM_BLK,),
      in_specs=[
          pl.BlockSpec((_MM_BLK, D), lambda i: (i, 0)),
          pl.BlockSpec((_MM_BLK, D), lambda i: (i, 0)),
          pl.BlockSpec((D, D), lambda i: (0, 0)),
          pl.BlockSpec((1, D), lambda i: (0, 0)),
      ],
      out_specs=pl.BlockSpec((_MM_BLK, D), lambda i: (i, 0)),
      out_shape=jax.ShapeDtypeStruct((N_PAD, D), jnp.float32),
  )(a0, a1, w, b.reshape(1, D))


def _pool_body(h_hbm, batch_hbm, out_hbm, batch_v, rows_v, out_v, sem):
  c = lax.axis_index("c")
  s = lax.axis_index("s")
  wid = c * NS + s
  g0 = wid * GPT

  pltpu.sync_copy(batch_hbm, batch_v)

  iota = lax.iota(jnp.int32, 16)

  def lower_bound(g):
    def body(_, lohi):
      lo, hi = lohi
      mid = (lo + hi) // 2
      less = batch_v[pl.ds(mid, 16)][0] < g
      return (jnp.where(less, mid + 1, lo), jnp.where(less, hi, mid))

    lo, _ = lax.fori_loop(0, 14, body,
                          (jnp.int32(0), jnp.int32(N_PAD)))
    return lo

  bounds = [lower_bound(g0 + t) for t in range(GPT + 1)]

  neg_inf = jnp.full((16,), -jnp.inf, jnp.float32)

  for t in range(GPT):
    start = bounds[t]
    end = bounds[t + 1]
    nch = (end - start + 15) // 16

    def chunk_body(ci, acc8, start=start, end=end):
      idx = jnp.minimum(start + ci * 16 + iota, end - 1)
      pltpu.async_copy(h_hbm.at[idx], rows_v, sem).wait()
      out = list(acc8)
      for k in range(8):
        m = out[k]
        for r in range(16):
          m = jnp.maximum(m, rows_v[r, pl.ds(k * 16, 16)])
        out[k] = m
      return tuple(out)

    acc8 = lax.fori_loop(0, nch, chunk_body,
                         tuple(neg_inf for _ in range(8)))
    for k in range(8):
      out_v[t, pl.ds(k * 16, 16)] = acc8[k]

  pltpu.sync_copy(out_v, out_hbm.at[pl.ds(g0, GPT)])


@functools.lru_cache(maxsize=None)
def _sc_kernels():
  mesh = plsc.VectorSubcoreMesh(
      core_axis_name="c", subcore_axis_name="s",
      num_cores=NC, num_subcores=NS)
  emb = pl.kernel(
      _emb_body,
      out_type=jax.ShapeDtypeStruct((N_PAD, D), jnp.float32),
      mesh=mesh,
      scratch_types=[
          pltpu.VMEM((TOK_CH, TCHUNK), jnp.int32),
          pltpu.VMEM((TOK_CH, TCHUNK), jnp.int32),
          pltpu.VMEM((_EMB_RING, TCHUNK, D), jnp.float32),
          pltpu.VMEM_SHARED((NPC, D), jnp.float32),
          pltpu.SemaphoreType.DMA((_EMB_RING,)),
          pltpu.SemaphoreType.DMA((_EMB_RING,)),
      ],
  )
  edge = pl.kernel(
      _edge_body,
      out_type=jax.ShapeDtypeStruct((NC, N_PAD, D), jnp.float32),
      mesh=mesh,
      scratch_types=[
          pltpu.VMEM((_EDGE_IBUF, 2, ECHUNK), jnp.int32),
          pltpu.VMEM((_EDGE_RBUF, ECHUNK, D), jnp.float32),
          pltpu.VMEM_SHARED((N_ACC, D), jnp.float32),
          pltpu.SemaphoreType.DMA((_EDGE_IBUF,)),
          pltpu.SemaphoreType.DMA((_EDGE_RBUF,)),
          pltpu.SemaphoreType.DMA((_EDGE_RBUF,)),
      ],
  )
  pool = pl.kernel(
      _pool_body,
      out_type=jax.ShapeDtypeStruct((G, D), jnp.float32),
      mesh=mesh,
      scratch_types=[
          pltpu.VMEM((N_PAD + 16,), jnp.int32),
          pltpu.VMEM((16, D), jnp.float32),
          pltpu.VMEM((GPT, D), jnp.float32),
          pltpu.SemaphoreType.DMA,
      ],
  )
  return emb, edge, pool


def kernel(x, edge_index, batch, emb_table, W0, b0, W1, b1):
  x = x.astype(jnp.int32)
  edge_index = edge_index.astype(jnp.int32)
  batch = batch.astype(jnp.int32)

  x_pad = jnp.zeros((N_PAD, L), jnp.int32).at[:N].set(x)
  tpad = TOK_PT - NPT * L
  tok = jnp.concatenate(
      [x_pad.reshape(NC, NS, NPT, L).transpose(0, 1, 3, 2).reshape(
          NC, NS, NPT * L),
       jnp.zeros((NC, NS, tpad), jnp.int32)], axis=2).reshape(
           NC, NS, TOK_CH, TCHUNK)
  dpat = jnp.concatenate([jnp.tile(jnp.arange(NPT, dtype=jnp.int32), L),
                          jnp.zeros((tpad,), jnp.int32)])
  emb_dst = (dpat[None, :]
             + (jnp.arange(NS, dtype=jnp.int32) * NPT)[:, None]).reshape(
                 NS, TOK_CH, TCHUNK)

  src = jnp.zeros((E_PAD,), jnp.int32).at[:E].set(edge_index[0])
  dst = jnp.full((E_PAD,), DUMP_ROW, jnp.int32).at[:E].set(edge_index[1])
  eidx = jnp.stack([src.reshape(NC, NS, EPT_CH, ECHUNK),
                    dst.reshape(NC, NS, EPT_CH, ECHUNK)], axis=3)

  batch_pad = jnp.full((N_PAD + 16,), G, jnp.int32).at[:N].set(batch)

  emb_k, edge_k, pool_k = _sc_kernels()
  h0 = emb_k(emb_table, tok, emb_dst)
  agg = edge_k(h0, eidx)
  h1 = _mm_relu(agg[0], agg[1], W0, b0)
  agg = edge_k(h1, eidx)
  h2 = _mm_relu(agg[0], agg[1], W1, b1)
  pooled = pool_k(h2, batch_pad)
  return pooled
